# baseline (speedup 1.0000x reference)
import jax
import jax.numpy as jnp
from jax.experimental import pallas as pl
from jax.experimental.pallas import tpu as pltpu

_BN = 80
_J = 12
_NV = 4
_NC = 8
_LANES = 128


def _kern(x_ref, cl_ref, ct_ref, ml_ref, mt_ref, sig_ref, out_ref):
    s = jnp.maximum(sig_ref[0, 0], 1e-10)
    h = -0.5 / (s * s)
    cl = cl_ref[...]
    ct = ct_ref[...]
    ml = ml_ref[...]
    mt = mt_ref[...]
    x = x_ref[...]

    ws = []
    denom = None
    for j in range(_J):
        a = cl[:, j:j + 1] - ml
        b = ct[:, j:j + 1] - mt
        wj = jnp.exp(a * a * h) * jnp.exp(b * b * h)
        ws.append(wj)
        denom = wj if denom is None else denom + wj
    for v in range(_NV):
        acc = None
        for j in range(_J):
            wn = ws[j] / denom
            xjv = x[:, j * (_NV * _NC) + v * _NC:
                    j * (_NV * _NC) + (v + 1) * _NC]
            xt = pltpu.repeat(xjv, _LANES // _NC, axis=1)
            t = wn * xt
            acc = t if acc is None else acc + t
        out_ref[:, v, :] = acc


def kernel(x, coords_lon, coords_lat, mus_lon, mus_lat, sigma):
    b, n, seq_ref, seq_in, nh_in = coords_lon.shape
    nv, nc = x.shape[-2], x.shape[-1]
    n_lon, n_lat = mus_lon.shape[0], mus_lat.shape[0]
    j = seq_in * nh_in
    lanes = n_lon * n_lat * nc

    x2 = x.reshape(n, j * nv * nc)
    cl2 = coords_lon.reshape(n, j)
    ct2 = coords_lat.reshape(n, j)
    ml128 = jnp.repeat(mus_lon, n_lat * nc).reshape(1, lanes)
    mt128 = jnp.tile(jnp.repeat(mus_lat, nc), n_lon).reshape(1, lanes)
    sig = jnp.asarray(sigma, jnp.float32).reshape(1, 1)

    grid = (n // _BN,)
    out = pl.pallas_call(
        _kern,
        grid=grid,
        in_specs=[
            pl.BlockSpec((_BN, j * nv * nc), lambda i: (i, 0)),
            pl.BlockSpec((_BN, j), lambda i: (i, 0)),
            pl.BlockSpec((_BN, j), lambda i: (i, 0)),
            pl.BlockSpec((1, lanes), lambda i: (0, 0)),
            pl.BlockSpec((1, lanes), lambda i: (0, 0)),
            pl.BlockSpec((1, 1), lambda i: (0, 0)),
        ],
        out_specs=pl.BlockSpec((_BN, nv, lanes), lambda i: (i, 0, 0)),
        out_shape=jax.ShapeDtypeStruct((n, nv, lanes), jnp.float32),
    )(x2, cl2, ct2, ml128, mt128, sig)
    return out.reshape(b, n, nv, n_lon, n_lat, nc)


# speedup vs baseline: 1.8159x; 1.8159x over previous
import jax
import jax.numpy as jnp
from jax.experimental import pallas as pl
from jax.experimental.pallas import tpu as pltpu

_BN = 80
_J = 12
_NV = 4
_NC = 8
_L128 = 128
_GROUP = 4
_NG = _J // _GROUP


def _kern(xg0_ref, xg1_ref, xg2_ref, t_ref, cl_ref, ct_ref, ml_ref, mt_ref,
          sig_ref, out_ref):
    s = jnp.maximum(sig_ref[0, 0], 1e-10)
    h = -0.5 / (s * s)
    cl = cl_ref[...]
    ct = ct_ref[...]
    ml = ml_ref[...]
    mt = mt_ref[...]
    tmat = t_ref[...]

    ws = []
    denom = None
    for j in range(_J):
        a = cl[:, j:j + 1] - ml
        b = ct[:, j:j + 1] - mt
        wj = jnp.exp(a * a * h) * jnp.exp(b * b * h)
        ws.append(wj)
        denom = wj if denom is None else denom + wj

    acc = None
    for g, xg_ref in enumerate((xg0_ref, xg1_ref, xg2_ref)):
        xt_g = jax.lax.dot_general(
            xg_ref[...], tmat, (((1,), (0,)), ((), ())),
            preferred_element_type=jnp.float32)
        for jl in range(_GROUP):
            j = g * _GROUP + jl
            wn = ws[j] / denom
            wn512 = pltpu.repeat(wn, _NV, axis=1)
            t = wn512 * xt_g[:, jl * 512:(jl + 1) * 512]
            acc = t if acc is None else acc + t
    out_ref[...] = acc


def _expand_matrix(j_group, nv, nc, n_mu):
    q = jnp.arange(j_group * nv * nc)
    p = jnp.arange(j_group * nv * n_mu * nc)
    qj, qv, qc = q // (nv * nc), (q // nc) % nv, q % nc
    pj, pv, pc = p // (nv * n_mu * nc), (p // (n_mu * nc)) % nv, p % nc
    eq = ((qj[:, None] == pj[None, :]) & (qv[:, None] == pv[None, :])
          & (qc[:, None] == pc[None, :]))
    return eq.astype(jnp.float32)


def kernel(x, coords_lon, coords_lat, mus_lon, mus_lat, sigma):
    b, n, seq_ref, seq_in, nh_in = coords_lon.shape
    nv, nc = x.shape[-2], x.shape[-1]
    n_lon, n_lat = mus_lon.shape[0], mus_lat.shape[0]
    j = seq_in * nh_in
    n_mu = n_lon * n_lat
    lanes = n_mu * nc

    x2 = x.reshape(n, j * nv * nc)
    cl2 = coords_lon.reshape(n, j)
    ct2 = coords_lat.reshape(n, j)
    ml128 = jnp.repeat(mus_lon, n_lat * nc).reshape(1, lanes)
    mt128 = jnp.tile(jnp.repeat(mus_lat, nc), n_lon).reshape(1, lanes)
    sig = jnp.asarray(sigma, jnp.float32).reshape(1, 1)
    tmat = _expand_matrix(_GROUP, nv, nc, n_mu)

    grid = (n // _BN,)
    xspecs = [pl.BlockSpec((_BN, _GROUP * nv * nc),
                           (lambda g: (lambda i: (i, g)))(g))
              for g in range(_NG)]
    out = pl.pallas_call(
        _kern,
        grid=grid,
        in_specs=xspecs + [
            pl.BlockSpec(tmat.shape, lambda i: (0, 0)),
            pl.BlockSpec((_BN, j), lambda i: (i, 0)),
            pl.BlockSpec((_BN, j), lambda i: (i, 0)),
            pl.BlockSpec((1, lanes), lambda i: (0, 0)),
            pl.BlockSpec((1, lanes), lambda i: (0, 0)),
            pl.BlockSpec((1, 1), lambda i: (0, 0)),
        ],
        out_specs=pl.BlockSpec((_BN, nv * n_mu * nc), lambda i: (i, 0)),
        out_shape=jax.ShapeDtypeStruct((n, nv * n_mu * nc), jnp.float32),
    )(x2, x2, x2, tmat, cl2, ct2, ml128, mt128, sig)
    return out.reshape(b, n, nv, n_lon, n_lat, nc)


# speedup vs baseline: 2.4102x; 1.3273x over previous
import jax
import jax.numpy as jnp
from jax.experimental import pallas as pl
from jax.experimental.pallas import tpu as pltpu

_BN = 200
_J = 12
_NV = 4
_NC = 8
_L128 = 128
_GROUP = 4
_NG = _J // _GROUP


def _kern(xg0_ref, xg1_ref, xg2_ref, t_ref, cl_ref, ct_ref, ml_ref, mt_ref,
          sig_ref, out_ref, ws_ref):
    s = jnp.maximum(sig_ref[0, 0], 1e-10)
    h = -0.5 / (s * s)
    cl = cl_ref[...]
    ct = ct_ref[...]
    ml = ml_ref[...]
    mt = mt_ref[...]

    denom = None
    for j in range(_J):
        a = cl[:, j:j + 1] - ml
        b = ct[:, j:j + 1] - mt
        wj = jnp.exp(a * a * h) * jnp.exp(b * b * h)
        ws_ref[j] = wj
        denom = wj if denom is None else denom + wj

    acc = None
    for g, xg_ref in enumerate((xg0_ref, xg1_ref, xg2_ref)):
        xg = xg_ref[...]
        for jl in range(_GROUP):
            j = g * _GROUP + jl
            xt = jax.lax.dot_general(
                xg, t_ref[jl], (((1,), (0,)), ((), ())),
                preferred_element_type=jnp.float32)
            wn = ws_ref[j] / denom
            wn512 = pltpu.repeat(wn, _NV, axis=1)
            t = wn512 * xt
            acc = t if acc is None else acc + t
    out_ref[...] = acc


def _expand_matrix(j_group, nv, nc, n_mu):
    q = jnp.arange(j_group * nv * nc)
    p = jnp.arange(nv * n_mu * nc)
    qj, qv, qc = q // (nv * nc), (q // nc) % nv, q % nc
    pv, pc = (p // (n_mu * nc)) % nv, p % nc
    jl = jnp.arange(j_group)
    eq = ((qj[None, :, None] == jl[:, None, None])
          & (qv[None, :, None] == pv[None, None, :])
          & (qc[None, :, None] == pc[None, None, :]))
    return eq.astype(jnp.float32)


def kernel(x, coords_lon, coords_lat, mus_lon, mus_lat, sigma):
    b, n, seq_ref, seq_in, nh_in = coords_lon.shape
    nv, nc = x.shape[-2], x.shape[-1]
    n_lon, n_lat = mus_lon.shape[0], mus_lat.shape[0]
    j = seq_in * nh_in
    n_mu = n_lon * n_lat
    lanes = n_mu * nc

    x2 = x.reshape(n, j * nv * nc)
    cl2 = coords_lon.reshape(n, j)
    ct2 = coords_lat.reshape(n, j)
    ml128 = jnp.repeat(mus_lon, n_lat * nc).reshape(1, lanes)
    mt128 = jnp.tile(jnp.repeat(mus_lat, nc), n_lon).reshape(1, lanes)
    sig = jnp.asarray(sigma, jnp.float32).reshape(1, 1)
    tmat = _expand_matrix(_GROUP, nv, nc, n_mu)

    grid = (n // _BN,)
    xspecs = [pl.BlockSpec((_BN, _GROUP * nv * nc),
                           (lambda g: (lambda i: (i, g)))(g))
              for g in range(_NG)]
    out = pl.pallas_call(
        _kern,
        grid=grid,
        in_specs=xspecs + [
            pl.BlockSpec(tmat.shape, lambda i: (0, 0, 0)),
            pl.BlockSpec((_BN, j), lambda i: (i, 0)),
            pl.BlockSpec((_BN, j), lambda i: (i, 0)),
            pl.BlockSpec((1, lanes), lambda i: (0, 0)),
            pl.BlockSpec((1, lanes), lambda i: (0, 0)),
            pl.BlockSpec((1, 1), lambda i: (0, 0)),
        ],
        out_specs=pl.BlockSpec((_BN, nv * n_mu * nc), lambda i: (i, 0)),
        out_shape=jax.ShapeDtypeStruct((n, nv * n_mu * nc), jnp.float32),
        scratch_shapes=[pltpu.VMEM((_J, _BN, lanes), jnp.float32)],
    )(x2, x2, x2, tmat, cl2, ct2, ml128, mt128, sig)
    return out.reshape(b, n, nv, n_lon, n_lat, nc)


# speedup vs baseline: 4.7312x; 1.9630x over previous
import jax
import jax.numpy as jnp
from jax.experimental import pallas as pl

_NL = 512
_J = 12
_NV = 4
_NC = 8
_NM = 16


def _kern(x_ref, cl_ref, ct_ref, ml_ref, mt_ref, sig_ref, out_ref):
    s = jnp.maximum(sig_ref[0, 0], 1e-10)
    h = -0.5 / (s * s)
    cl = cl_ref[...]
    ct = ct_ref[...]
    ml = ml_ref[...]
    mt = mt_ref[...]

    ws = []
    denom = None
    for j in range(_J):
        a = cl[j:j + 1, :] - ml
        b = ct[j:j + 1, :] - mt
        wj = jnp.exp(a * a * h) * jnp.exp(b * b * h)
        ws.append(wj)
        denom = wj if denom is None else denom + wj

    acc = None
    for j in range(_J):
        wn = ws[j] / denom
        xj = x_ref[j * 32:(j + 1) * 32, :].reshape(_NV, _NC, -1)
        t = wn[None, :, None, :] * xj[:, None, :, :]
        acc = t if acc is None else acc + t
    out_ref[...] = acc.reshape(_NV * _NM * _NC, -1)


def kernel(x, coords_lon, coords_lat, mus_lon, mus_lat, sigma):
    b, n, seq_ref, seq_in, nh_in = coords_lon.shape
    nv, nc = x.shape[-2], x.shape[-1]
    n_lon, n_lat = mus_lon.shape[0], mus_lat.shape[0]
    j = seq_in * nh_in
    n_mu = n_lon * n_lat

    xt = x.reshape(n, j * nv * nc).T
    clt = coords_lon.reshape(n, j).T
    ctt = coords_lat.reshape(n, j).T
    ml16 = jnp.repeat(mus_lon, n_lat).reshape(n_mu, 1)
    mt16 = jnp.tile(mus_lat, n_lon).reshape(n_mu, 1)
    sig = jnp.asarray(sigma, jnp.float32).reshape(1, 1)

    grid = (pl.cdiv(n, _NL),)
    out = pl.pallas_call(
        _kern,
        grid=grid,
        in_specs=[
            pl.BlockSpec((j * nv * nc, _NL), lambda i: (0, i)),
            pl.BlockSpec((j, _NL), lambda i: (0, i)),
            pl.BlockSpec((j, _NL), lambda i: (0, i)),
            pl.BlockSpec((n_mu, 1), lambda i: (0, 0)),
            pl.BlockSpec((n_mu, 1), lambda i: (0, 0)),
            pl.BlockSpec((1, 1), lambda i: (0, 0)),
        ],
        out_specs=pl.BlockSpec((nv * n_mu * nc, _NL), lambda i: (0, i)),
        out_shape=jax.ShapeDtypeStruct((nv * n_mu * nc, n), jnp.float32),
    )(xt, clt, ctt, ml16, mt16, sig)
    return out.reshape(nv, n_lon, n_lat, nc, n).transpose(4, 0, 1, 2, 3)[None]
